# baseline (speedup 1.0000x reference)
import functools

import jax
import jax.numpy as jnp
from jax import lax
from jax.experimental import pallas as pl
from jax.experimental.pallas import tpu as pltpu
from jax.experimental.pallas import tpu_sc as plsc

_K = 64
_NBUF = 5


def _mesh():
    return plsc.VectorSubcoreMesh(core_axis_name="c", subcore_axis_name="s")


def _nchunks(e):
    c = -(-e // (32 * _K))
    return -(-c // _NBUF) * _NBUF


@functools.lru_cache(maxsize=None)
def _deg_kernel(n, e):
    nc = _nchunks(e)
    lpw = n // 16

    @functools.partial(
        pl.kernel,
        out_type=jax.ShapeDtypeStruct((2 * n,), jnp.float32),
        mesh=_mesh(),
        compiler_params=pltpu.CompilerParams(needs_layout_passes=False),
        scratch_types=[
            [pltpu.VMEM((2, _K), jnp.int32) for _ in range(_NBUF)],
            pltpu.VMEM((n,), jnp.float32),
            pltpu.VMEM((lpw,), jnp.float32),
            pltpu.VMEM((lpw,), jnp.float32),
            pltpu.VMEM_SHARED((16 * n,), jnp.float32),
            [pltpu.SemaphoreType.DMA for _ in range(_NBUF)],
        ],
    )
    def k(sd_hbm, zer_hbm, degp_hbm, sd, dgrid, tmp, red, sacc, si):
        cid = lax.axis_index("c")
        sid = lax.axis_index("s")
        w = cid * 16 + sid
        pltpu.sync_copy(zer_hbm, dgrid)

        def i_start(c, j):
            pltpu.async_copy(sd_hbm.at[w * nc + c], sd[j], si[j])

        def i_wait(j):
            pltpu.make_async_copy(sd_hbm.at[0], sd[j], si[j]).wait()

        for j in range(_NBUF):
            i_start(j, j)
        ones16 = jnp.ones((16,), jnp.float32)

        def body(i, carry):
            c0 = _NBUF * i
            for j in range(_NBUF):
                i_wait(j)
                for m in range(_K // 16):
                    dstv = sd[j][1, pl.ds(m * 16, 16)]
                    plsc.addupdate_scatter(dgrid, [dstv], ones16)

                @pl.when(i + 1 < nc // _NBUF)
                def _(j=j):
                    i_start(c0 + _NBUF + j, j)

            return carry

        lax.fori_loop(0, nc // _NBUF, body, 0)
        pltpu.sync_copy(dgrid, sacc.at[pl.ds(sid * n, n)])
        plsc.subcore_barrier()
        pltpu.sync_copy(sacc.at[pl.ds(sid * lpw, lpw)], red)

        def rbody(p, carry):
            pltpu.sync_copy(sacc.at[pl.ds(p * n + sid * lpw, lpw)], tmp)
            for t in range(lpw // 16):
                s = pl.ds(t * 16, 16)
                red[s] = red[s] + tmp[s]
            return carry

        lax.fori_loop(1, 16, rbody, 0)
        pltpu.sync_copy(red, degp_hbm.at[pl.ds(cid * n + sid * lpw, lpw)])

    return k


@functools.lru_cache(maxsize=None)
def _edge_kernel(n, e, d):
    nc = _nchunks(e)
    rpw = n // 16

    @functools.partial(
        pl.kernel,
        out_type=jax.ShapeDtypeStruct((2 * n, d), jnp.float32),
        mesh=_mesh(),
        scratch_types=[
            [pltpu.VMEM((2, _K), jnp.int32) for _ in range(_NBUF)],
            [pltpu.VMEM((_K, d), jnp.float32) for _ in range(_NBUF)],
            pltpu.VMEM_SHARED((n, d), jnp.float32),
            [pltpu.SemaphoreType.DMA for _ in range(_NBUF)],
            [pltpu.SemaphoreType.DMA for _ in range(_NBUF)],
            [pltpu.SemaphoreType.DMA for _ in range(_NBUF)],
        ],
    )
    def k(t_hbm, sd_hbm, zer_hbm, acc_hbm,
          sd, rows, acc, si, sg, ss):
        cid = lax.axis_index("c")
        sid = lax.axis_index("s")
        w = cid * 16 + sid
        r0 = sid * rpw

        @pl.when(cid == 0)
        def _():
            pltpu.sync_copy(t_hbm.at[pl.ds(r0, rpw)], acc.at[pl.ds(r0, rpw)])

        @pl.when(cid != 0)
        def _():
            pltpu.sync_copy(zer_hbm.at[pl.ds(r0, rpw)], acc.at[pl.ds(r0, rpw)])

        def i_start(c, j):
            pltpu.async_copy(sd_hbm.at[w * nc + c], sd[j], si[j])

        def i_wait(j):
            pltpu.make_async_copy(sd_hbm.at[0], sd[j], si[j]).wait()

        def g_start(j):
            pltpu.async_copy(t_hbm.at[sd[j].at[0]], rows[j], sg[j])

        def g_wait(j):
            pltpu.make_async_copy(t_hbm.at[sd[j].at[0]], rows[j], sg[j]).wait()

        def s_start(j):
            pltpu.async_copy(rows[j], acc.at[sd[j].at[1]], ss[j], add=True)

        def s_wait(j):
            pltpu.make_async_copy(rows[j], acc.at[sd[j].at[1]], ss[j]).wait()

        for j in range(_NBUF):
            i_start(j, j)
        plsc.subcore_barrier()
        for j in range(_NBUF):
            i_wait(j)
            g_start(j)

        def body(i, carry):
            c0 = _NBUF * i
            for j in range(_NBUF):
                g_wait(j)
                s_start(j)
            for j in range(_NBUF):
                s_wait(j)

                @pl.when(i + 1 < nc // _NBUF)
                def _(j=j):
                    i_start(c0 + _NBUF + j, j)
                    i_wait(j)
                    g_start(j)

            return carry

        lax.fori_loop(0, nc // _NBUF, body, 0)
        plsc.subcore_barrier()
        pltpu.sync_copy(acc.at[pl.ds(r0, rpw)],
                        acc_hbm.at[pl.ds(cid * n + r0, rpw)])

    return k


def _prep_body(degp_ref, x_ref, t1_ref):
    deg = degp_ref[0] + degp_ref[1] + 1.0
    dinv = lax.rsqrt(deg)
    t1_ref[...] = x_ref[...] * jnp.broadcast_to(dinv, x_ref.shape)


@functools.lru_cache(maxsize=None)
def _prep_kernel(n, d, blk):
    grid = n // blk
    return pl.pallas_call(
        _prep_body,
        grid=(grid,),
        in_specs=[
            pl.BlockSpec((2, blk, 1), lambda i: (0, i, 0)),
            pl.BlockSpec((blk, d), lambda i: (i, 0)),
        ],
        out_specs=pl.BlockSpec((blk, d), lambda i: (i, 0)),
        out_shape=jax.ShapeDtypeStruct((n, d), jnp.float32),
    )


def _layer_body(acc_ref, degp_ref, w_ref, b_ref, out_ref, *, mid):
    deg = degp_ref[0] + degp_ref[1] + 1.0
    dinvr = jnp.broadcast_to(lax.rsqrt(deg), out_ref.shape)
    a = (acc_ref[0] + acc_ref[1]) * dinvr
    h = jnp.dot(a, w_ref[...], preferred_element_type=jnp.float32) + b_ref[...]
    if mid:
        h = jnp.maximum(h, 0.0) * dinvr
    out_ref[...] = h


@functools.lru_cache(maxsize=None)
def _layer_kernel(n, d, blk, mid):
    grid = n // blk
    return pl.pallas_call(
        functools.partial(_layer_body, mid=mid),
        grid=(grid,),
        in_specs=[
            pl.BlockSpec((2, blk, d), lambda i: (0, i, 0)),
            pl.BlockSpec((2, blk, 1), lambda i: (0, i, 0)),
            pl.BlockSpec((d, d), lambda i: (0, 0)),
            pl.BlockSpec((1, d), lambda i: (0, 0)),
        ],
        out_specs=pl.BlockSpec((blk, d), lambda i: (i, 0)),
        out_shape=jax.ShapeDtypeStruct((n, d), jnp.float32),
    )


def kernel(x, edge_index, W1, b1, W2, b2):
    n, d = x.shape
    e = edge_index.shape[1]
    np_ = ((n + 256) // 256) * 256
    x_p = jnp.pad(x, ((0, np_ - n), (0, 0)))
    nc = _nchunks(e)
    ep = 32 * nc * _K
    pad_idx = n + (jnp.arange(ep - e, dtype=edge_index.dtype) % (np_ - n))
    ei = jnp.concatenate([edge_index, jnp.tile(pad_idx, (2, 1))], axis=1)
    sd = jnp.transpose(ei.reshape(2, 32 * nc, _K), (1, 0, 2))
    zern = jnp.zeros((np_,), jnp.float32)
    zerd = jnp.zeros((np_, d), jnp.float32)
    blk = np_ // 16

    degp = _deg_kernel(np_, e)(sd, zern).reshape(2, np_, 1)
    t1 = _prep_kernel(np_, d, blk)(degp, x_p)
    acc1 = _edge_kernel(np_, e, d)(t1, sd, zerd).reshape(2, np_, d)
    t2 = _layer_kernel(np_, d, blk, True)(acc1, degp, W1, b1.reshape(1, d))
    acc2 = _edge_kernel(np_, e, d)(t2, sd, zerd).reshape(2, np_, d)
    out = _layer_kernel(np_, d, blk, False)(acc2, degp, W2, b2.reshape(1, d))
    return out[:n]


# speedup vs baseline: 1.1651x; 1.1651x over previous
import functools

import jax
import jax.numpy as jnp
from jax import lax
from jax.experimental import pallas as pl
from jax.experimental.pallas import tpu as pltpu
from jax.experimental.pallas import tpu_sc as plsc

_K = 128
_NBUF = 3


def _mesh():
    return plsc.VectorSubcoreMesh(core_axis_name="c", subcore_axis_name="s")


def _nchunks(e):
    c = -(-e // (32 * _K))
    return -(-c // _NBUF) * _NBUF


@functools.lru_cache(maxsize=None)
def _deg_kernel(n, e):
    nc = _nchunks(e)
    lpw = n // 16

    @functools.partial(
        pl.kernel,
        out_type=jax.ShapeDtypeStruct((2 * n,), jnp.float32),
        mesh=_mesh(),
        compiler_params=pltpu.CompilerParams(needs_layout_passes=False),
        scratch_types=[
            [pltpu.VMEM((2, _K), jnp.int32) for _ in range(_NBUF)],
            pltpu.VMEM((n,), jnp.float32),
            pltpu.VMEM((lpw,), jnp.float32),
            pltpu.VMEM((lpw,), jnp.float32),
            pltpu.VMEM_SHARED((16 * n,), jnp.float32),
            [pltpu.SemaphoreType.DMA for _ in range(_NBUF)],
        ],
    )
    def k(sd_hbm, zer_hbm, degp_hbm, sd, dgrid, tmp, red, sacc, si):
        cid = lax.axis_index("c")
        sid = lax.axis_index("s")
        w = cid * 16 + sid
        pltpu.sync_copy(zer_hbm, dgrid)

        def i_start(c, j):
            pltpu.async_copy(sd_hbm.at[w * nc + c], sd[j], si[j])

        def i_wait(j):
            pltpu.make_async_copy(sd_hbm.at[0], sd[j], si[j]).wait()

        for j in range(_NBUF):
            i_start(j, j)
        ones16 = jnp.ones((16,), jnp.float32)

        def body(i, carry):
            c0 = _NBUF * i
            for j in range(_NBUF):
                i_wait(j)
                for m in range(_K // 16):
                    dstv = sd[j][1, pl.ds(m * 16, 16)]
                    plsc.addupdate_scatter(dgrid, [dstv], ones16)

                @pl.when(i + 1 < nc // _NBUF)
                def _(j=j):
                    i_start(c0 + _NBUF + j, j)

            return carry

        lax.fori_loop(0, nc // _NBUF, body, 0)
        pltpu.sync_copy(dgrid, sacc.at[pl.ds(sid * n, n)])
        plsc.subcore_barrier()
        pltpu.sync_copy(sacc.at[pl.ds(sid * lpw, lpw)], red)

        def rbody(p, carry):
            pltpu.sync_copy(sacc.at[pl.ds(p * n + sid * lpw, lpw)], tmp)
            for t in range(lpw // 16):
                s = pl.ds(t * 16, 16)
                red[s] = red[s] + tmp[s]
            return carry

        lax.fori_loop(1, 16, rbody, 0)
        pltpu.sync_copy(red, degp_hbm.at[pl.ds(cid * n + sid * lpw, lpw)])

    return k


@functools.lru_cache(maxsize=None)
def _edge_kernel(n, e, d):
    nc = _nchunks(e)
    rpw = n // 16

    @functools.partial(
        pl.kernel,
        out_type=jax.ShapeDtypeStruct((2 * n, d), jnp.float32),
        mesh=_mesh(),
        scratch_types=[
            [pltpu.VMEM((2, _K), jnp.int32) for _ in range(_NBUF)],
            [pltpu.VMEM((_K, d), jnp.float32) for _ in range(_NBUF)],
            pltpu.VMEM_SHARED((n, d), jnp.float32),
            [pltpu.SemaphoreType.DMA for _ in range(_NBUF)],
            [pltpu.SemaphoreType.DMA for _ in range(_NBUF)],
            [pltpu.SemaphoreType.DMA for _ in range(_NBUF)],
        ],
    )
    def k(t_hbm, sd_hbm, zer_hbm, acc_hbm,
          sd, rows, acc, si, sg, ss):
        cid = lax.axis_index("c")
        sid = lax.axis_index("s")
        w = cid * 16 + sid
        r0 = sid * rpw

        @pl.when(cid == 0)
        def _():
            pltpu.sync_copy(t_hbm.at[pl.ds(r0, rpw)], acc.at[pl.ds(r0, rpw)])

        @pl.when(cid != 0)
        def _():
            pltpu.sync_copy(zer_hbm.at[pl.ds(r0, rpw)], acc.at[pl.ds(r0, rpw)])

        def i_start(c, j):
            pltpu.async_copy(sd_hbm.at[w * nc + c], sd[j], si[j])

        def i_wait(j):
            pltpu.make_async_copy(sd_hbm.at[0], sd[j], si[j]).wait()

        def g_start(j):
            pltpu.async_copy(t_hbm.at[sd[j].at[0]], rows[j], sg[j])

        def g_wait(j):
            pltpu.make_async_copy(t_hbm.at[sd[j].at[0]], rows[j], sg[j]).wait()

        def s_start(j):
            pltpu.async_copy(rows[j], acc.at[sd[j].at[1]], ss[j], add=True)

        def s_wait(j):
            pltpu.make_async_copy(rows[j], acc.at[sd[j].at[1]], ss[j]).wait()

        for j in range(_NBUF):
            i_start(j, j)
        plsc.subcore_barrier()
        for j in range(_NBUF):
            i_wait(j)
            g_start(j)

        def body(i, carry):
            c0 = _NBUF * i
            for j in range(_NBUF):
                g_wait(j)
                s_start(j)
            for j in range(_NBUF):
                s_wait(j)

                @pl.when(i + 1 < nc // _NBUF)
                def _(j=j):
                    i_start(c0 + _NBUF + j, j)
                    i_wait(j)
                    g_start(j)

            return carry

        lax.fori_loop(0, nc // _NBUF, body, 0)
        plsc.subcore_barrier()
        pltpu.sync_copy(acc.at[pl.ds(r0, rpw)],
                        acc_hbm.at[pl.ds(cid * n + r0, rpw)])

    return k


def _prep_body(degp_ref, x_ref, t1_ref):
    deg = degp_ref[0] + degp_ref[1] + 1.0
    dinv = lax.rsqrt(deg)
    t1_ref[...] = x_ref[...] * jnp.broadcast_to(dinv, x_ref.shape)


@functools.lru_cache(maxsize=None)
def _prep_kernel(n, d, blk):
    grid = n // blk
    return pl.pallas_call(
        _prep_body,
        grid=(grid,),
        in_specs=[
            pl.BlockSpec((2, blk, 1), lambda i: (0, i, 0)),
            pl.BlockSpec((blk, d), lambda i: (i, 0)),
        ],
        out_specs=pl.BlockSpec((blk, d), lambda i: (i, 0)),
        out_shape=jax.ShapeDtypeStruct((n, d), jnp.float32),
    )


def _layer_body(acc_ref, degp_ref, w_ref, b_ref, out_ref, *, mid):
    deg = degp_ref[0] + degp_ref[1] + 1.0
    dinvr = jnp.broadcast_to(lax.rsqrt(deg), out_ref.shape)
    a = (acc_ref[0] + acc_ref[1]) * dinvr
    h = jnp.dot(a, w_ref[...], preferred_element_type=jnp.float32) + b_ref[...]
    if mid:
        h = jnp.maximum(h, 0.0) * dinvr
    out_ref[...] = h


@functools.lru_cache(maxsize=None)
def _layer_kernel(n, d, blk, mid):
    grid = n // blk
    return pl.pallas_call(
        functools.partial(_layer_body, mid=mid),
        grid=(grid,),
        in_specs=[
            pl.BlockSpec((2, blk, d), lambda i: (0, i, 0)),
            pl.BlockSpec((2, blk, 1), lambda i: (0, i, 0)),
            pl.BlockSpec((d, d), lambda i: (0, 0)),
            pl.BlockSpec((1, d), lambda i: (0, 0)),
        ],
        out_specs=pl.BlockSpec((blk, d), lambda i: (i, 0)),
        out_shape=jax.ShapeDtypeStruct((n, d), jnp.float32),
    )


def kernel(x, edge_index, W1, b1, W2, b2):
    n, d = x.shape
    e = edge_index.shape[1]
    np_ = ((n + 128) // 128) * 128
    npr = ((n + 256) // 256) * 256
    x_p = jnp.pad(x, ((0, np_ - n), (0, 0)))
    nc = _nchunks(e)
    ep = 32 * nc * _K
    pad_idx = n + (jnp.arange(ep - e, dtype=edge_index.dtype) % (np_ - n))
    ei = jnp.concatenate([edge_index, jnp.tile(pad_idx, (2, 1))], axis=1)
    sd = jnp.transpose(ei.reshape(2, 32 * nc, _K), (1, 0, 2))
    zern = jnp.zeros((npr,), jnp.float32)
    zerd = jnp.zeros((np_, d), jnp.float32)
    blk = np_ // 16

    degp = _deg_kernel(npr, e)(sd, zern).reshape(2, npr, 1)[:, :np_]
    t1 = _prep_kernel(np_, d, blk)(degp, x_p)
    acc1 = _edge_kernel(np_, e, d)(t1, sd, zerd).reshape(2, np_, d)
    t2 = _layer_kernel(np_, d, blk, True)(acc1, degp, W1, b1.reshape(1, d))
    acc2 = _edge_kernel(np_, e, d)(t2, sd, zerd).reshape(2, np_, d)
    out = _layer_kernel(np_, d, blk, False)(acc2, degp, W2, b2.reshape(1, d))
    return out[:n]
